# Initial kernel scaffold; baseline (speedup 1.0000x reference)
import jax
import jax.numpy as jnp
from jax.experimental import pallas as pl


def kernel(feats, keys, idxs):
    raise NotImplementedError("write your pallas kernel here")


# speedup vs baseline: 2.2925x; 2.2925x over previous
import functools

import jax
import jax.numpy as jnp
from jax import lax
from jax.experimental import pallas as pl
from jax.experimental.pallas import tpu as pltpu
from jax.experimental.pallas import tpu_sc as plsc

DIM = 128
M = 100000
B = 4096
TEMP = 0.07
MBLK = 2048

def _sc_gather(keys, idxs):
    info = plsc.get_sparse_core_info()
    nc, ns = info.num_cores, info.num_subcores
    nw = nc * ns
    b_per_w = B // nw
    mesh = plsc.VectorSubcoreMesh(core_axis_name="c", subcore_axis_name="s")

    @functools.partial(
        pl.kernel,
        mesh=mesh,
        out_type=jax.ShapeDtypeStruct((B, DIM), jnp.float32),
        scratch_types=[
            pltpu.VMEM((b_per_w,), jnp.int32),
            pltpu.VMEM((b_per_w, DIM), jnp.float32),
            pltpu.SemaphoreType.DMA,
        ],
    )
    def gather_kernel(keys_hbm, idx_hbm, out_hbm, idx_v, rows_v, sem):
        wid = lax.axis_index("s") * nc + lax.axis_index("c")
        base = wid * b_per_w
        pltpu.sync_copy(idx_hbm.at[pl.ds(base, b_per_w)], idx_v)
        pltpu.async_copy(keys_hbm.at[idx_v], rows_v, sem).wait()
        pltpu.sync_copy(rows_v, out_hbm.at[pl.ds(base, b_per_w)])

    return gather_kernel(keys, idxs)


def _l2n(x):
    n = jnp.sqrt(jnp.sum(x * x, axis=1, keepdims=True))
    return x / jnp.maximum(n, 1e-12)


def _loss_body(feats_ref, keys_ref, tgt_ref, out_ref, fn_scr, s_scr):
    j = pl.program_id(0)
    nj = pl.num_programs(0)
    c = jnp.float32(1.0 / TEMP)

    @pl.when(j == 0)
    def _init():
        fn_scr[...] = _l2n(feats_ref[...])
        s_scr[...] = jnp.zeros_like(s_scr)

    fn = fn_scr[...]
    kn = _l2n(keys_ref[...])
    logits = lax.dot_general(
        fn, kn, (((1,), (1,)), ((), ())), preferred_element_type=jnp.float32
    ) * c
    col = j * MBLK + lax.broadcasted_iota(jnp.int32, (1, MBLK), 1)
    contrib = jnp.where(col < M, jnp.exp(logits - c), 0.0)
    s_scr[...] += jnp.sum(contrib, axis=1, keepdims=True)

    @pl.when(j == nj - 1)
    def _fin():
        tkn = _l2n(tgt_ref[...])
        tgt = jnp.sum(fn * tkn, axis=1, keepdims=True) * c
        lse = c + jnp.log(s_scr[...])
        out_ref[0, 0] = jnp.sum(lse - tgt) * jnp.float32(1.0 / B)


def kernel(feats, keys, idxs):
    tgt_keys = _sc_gather(keys, idxs.astype(jnp.int32))
    grid = (M + MBLK - 1) // MBLK
    loss = pl.pallas_call(
        _loss_body,
        grid=(grid,),
        in_specs=[
            pl.BlockSpec((B, DIM), lambda j: (0, 0)),
            pl.BlockSpec((MBLK, DIM), lambda j: (j, 0)),
            pl.BlockSpec((B, DIM), lambda j: (0, 0)),
        ],
        out_specs=pl.BlockSpec(memory_space=pltpu.SMEM),
        out_shape=jax.ShapeDtypeStruct((1, 1), jnp.float32),
        scratch_shapes=[
            pltpu.VMEM((B, DIM), jnp.float32),
            pltpu.VMEM((B, 1), jnp.float32),
        ],
        compiler_params=pltpu.CompilerParams(
            dimension_semantics=("arbitrary",),
        ),
    )(feats, keys, tgt_keys)
    return loss[0, 0]
